# baseline (speedup 1.0000x reference)
import jax
import jax.numpy as jnp
from jax.experimental import pallas as pl
from jax.experimental.pallas import tpu as pltpu

_EPS = 1e-8
_H = 128
_W = 128
_C = 256
_B = 4
_C_CHUNK = 64
_NUM_CC = _C // _C_CHUNK
_TILE_R = 16
_NUM_T = _H // _TILE_R

_SHIFTS = [(1, 0), (2, 0)] + [(di, dj) for dj in (1, 2) for di in range(-2, 3)]


def _roll2(x, di, dj):
    if di:
        x = jnp.roll(x, -di, axis=x.ndim - 2)
    if dj:
        x = jnp.roll(x, -dj, axis=x.ndim - 1)
    return x


def _cbl_kernel(seg_ref, gtb_ref, f_ref, out_ref, d_acc, n_acc, pad1, pad2,
                s_acc):
    b = pl.program_id(0)
    cc = pl.program_id(1)

    @pl.when(cc == 0)
    def _reset_image_acc():
        d_acc[...] = jnp.zeros_like(d_acc)
        n_acc[...] = jnp.zeros_like(n_acc)

    @pl.when(jnp.logical_and(b == 0, cc == 0))
    def _reset_global_acc():
        s_acc[0] = jnp.float32(0.0)
        s_acc[1] = jnp.float32(0.0)

    f = f_ref[0]
    zrows = jnp.zeros((_C_CHUNK, 2, _W), jnp.float32)
    r1 = jnp.roll(f, -1, axis=2)
    pad1[:, 2:_H + 2, :] = r1
    pad1[:, 0:2, :] = zrows
    pad1[:, _H + 2:_H + 4, :] = zrows
    r2 = jnp.roll(r1, -1, axis=2)
    pad2[:, 2:_H + 2, :] = r2
    pad2[:, 0:2, :] = zrows
    pad2[:, _H + 2:_H + 4, :] = zrows

    for t in range(_NUM_T):
        base = t * _TILE_R

        def body(c, carry):
            accs, nacc = carry
            left = f_ref[0, c, base:base + _TILE_R, :]
            nacc = nacc + left * left
            new = []
            for s_idx, (di, dj) in enumerate(_SHIFTS):
                if dj == 0:
                    lo = base + di
                    n_valid = min(_H - lo, _TILE_R)
                    right = f_ref[0, c, lo:lo + n_valid, :]
                    if n_valid == _TILE_R:
                        new.append(accs[s_idx] + left * right)
                    else:
                        upd = accs[s_idx][:n_valid] + left[:n_valid] * right
                        new.append(
                            jnp.concatenate([upd, accs[s_idx][n_valid:]],
                                            axis=0))
                else:
                    src = pad1 if dj == 1 else pad2
                    right = src[c, base + 2 + di:base + 2 + di + _TILE_R, :]
                    new.append(accs[s_idx] + left * right)
            return tuple(new), nacc

        zero_tile = jnp.zeros((_TILE_R, _W), jnp.float32)
        accs, nacc = jax.lax.fori_loop(
            0, _C_CHUNK, body,
            (tuple(zero_tile for _ in _SHIFTS), zero_tile))
        n_acc[base:base + _TILE_R, :] += nacc
        for s_idx in range(len(_SHIFTS)):
            d_acc[s_idx, base:base + _TILE_R, :] += accs[s_idx]

    @pl.when(cc == _NUM_CC - 1)
    def _finalize_image():
        seg = seg_ref[0]
        gtb = gtb_ref[0]
        seg = jnp.where(seg == 255, 0, seg)
        gtb = jnp.where(gtb == 255, 0, gtb)
        r = jax.lax.broadcasted_iota(jnp.int32, (_H, _W), 0)
        c = jax.lax.broadcasted_iota(jnp.int32, (_H, _W), 1)
        interior = (r >= 2) & (r <= _H - 3) & (c >= 2) & (c <= _W - 3)
        maskf = jnp.where((gtb * seg > 0) & interior,
                          jnp.float32(1.0), jnp.float32(0.0))
        inv = jnp.float32(1.0) / jnp.maximum(jnp.sqrt(n_acc[...]),
                                             jnp.float32(_EPS))
        s_total = jnp.float32(0.0)
        for s_idx, (di, dj) in enumerate(_SHIFTS):
            inv_nb = _roll2(inv, di, dj)
            sim = d_acc[s_idx] * inv * inv_nb
            seg_nb = _roll2(seg, di, dj)
            slab = jnp.where(seg == seg_nb, jnp.float32(1.0), jnp.float32(0.0))
            w = maskf + _roll2(maskf, di, dj)
            s_total += jnp.sum(w * (sim - slab) ** 2)
        count = jnp.sum(maskf)
        valid = count >= jnp.float32(1.0)
        contrib = jnp.where(
            valid,
            s_total / (jnp.maximum(count, jnp.float32(1.0)) * jnp.float32(24.0)),
            jnp.float32(0.0))
        s_acc[0] += contrib
        s_acc[1] += jnp.where(valid, jnp.float32(1.0), jnp.float32(0.0))

        @pl.when(b == _B - 1)
        def _finalize_total():
            tot = s_acc[0] / jnp.maximum(s_acc[1], jnp.float32(1.0))
            tot = jnp.where(s_acc[1] == jnp.float32(0.0), jnp.float32(0.0), tot)
            tot = jnp.where(jnp.isnan(tot), jnp.float32(0.0), tot)
            out_ref[...] = jnp.full((1, 1), tot, dtype=jnp.float32)


def kernel(er_input, seg_label, gt_boundary_seg):
    seg_ds = seg_label[:, ::4, ::4]
    gtb_ds = gt_boundary_seg[:, ::4, ::4]

    out = pl.pallas_call(
        _cbl_kernel,
        grid=(_B, _NUM_CC),
        in_specs=[
            pl.BlockSpec((1, _H, _W), lambda b, cc: (b, 0, 0)),
            pl.BlockSpec((1, _H, _W), lambda b, cc: (b, 0, 0)),
            pl.BlockSpec((1, _C_CHUNK, _H, _W), lambda b, cc: (b, cc, 0, 0)),
        ],
        out_specs=pl.BlockSpec((1, 1), lambda b, cc: (0, 0)),
        out_shape=jax.ShapeDtypeStruct((1, 1), jnp.float32),
        scratch_shapes=[
            pltpu.VMEM((len(_SHIFTS), _H, _W), jnp.float32),
            pltpu.VMEM((_H, _W), jnp.float32),
            pltpu.VMEM((_C_CHUNK, _H + 4, _W), jnp.float32),
            pltpu.VMEM((_C_CHUNK, _H + 4, _W), jnp.float32),
            pltpu.SMEM((2,), jnp.float32),
        ],
    )(seg_ds, gtb_ds, er_input)
    return out.reshape(())


# speedup vs baseline: 1.2428x; 1.2428x over previous
import jax
import jax.numpy as jnp
from jax.experimental import pallas as pl
from jax.experimental.pallas import tpu as pltpu

_EPS = 1e-8
_H = 128
_W = 128
_C = 256
_B = 4
_C_CHUNK = 64
_NUM_CC = _C // _C_CHUNK
_TILE_R = 16
_NUM_T = _H // _TILE_R

_SHIFTS = [(1, 0), (2, 0)] + [(di, dj) for dj in (1, 2) for di in range(-2, 3)]


def _roll2(x, di, dj):
    if di:
        x = jnp.roll(x, -di, axis=x.ndim - 2)
    if dj:
        x = jnp.roll(x, -dj, axis=x.ndim - 1)
    return x


def _cbl_kernel(seg_ref, gtb_ref, f_ref, out_ref, d_acc, n_acc, pad1, pad2,
                fchunk, s_acc):
    b = pl.program_id(0)
    cc = pl.program_id(1)

    @pl.when(cc == 0)
    def _reset_image_acc():
        d_acc[...] = jnp.zeros_like(d_acc)
        n_acc[...] = jnp.zeros_like(n_acc)

    @pl.when(jnp.logical_and(b == 0, cc == 0))
    def _reset_global_acc():
        s_acc[0] = jnp.float32(0.0)
        s_acc[1] = jnp.float32(0.0)

    f = f_ref[0].astype(jnp.float32)
    fchunk[...] = f
    zrows = jnp.zeros((_C_CHUNK, 2, _W), jnp.float32)
    r1 = jnp.roll(f, -1, axis=2)
    pad1[:, 2:_H + 2, :] = r1
    pad1[:, 0:2, :] = zrows
    pad1[:, _H + 2:_H + 4, :] = zrows
    r2 = jnp.roll(r1, -1, axis=2)
    pad2[:, 2:_H + 2, :] = r2
    pad2[:, 0:2, :] = zrows
    pad2[:, _H + 2:_H + 4, :] = zrows

    for t in range(_NUM_T):
        base = t * _TILE_R
        zero_tile = jnp.zeros((_TILE_R, _W), jnp.float32)
        accs = [zero_tile for _ in _SHIFTS]
        nacc = zero_tile
        for c in range(_C_CHUNK):
            left = fchunk[c, base:base + _TILE_R, :]
            nacc = nacc + left * left
            for s_idx, (di, dj) in enumerate(_SHIFTS):
                if dj == 0:
                    lo = base + di
                    n_valid = min(_H - lo, _TILE_R)
                    right = fchunk[c, lo:lo + n_valid, :]
                    if n_valid == _TILE_R:
                        accs[s_idx] = accs[s_idx] + left * right
                    else:
                        upd = accs[s_idx][:n_valid] + left[:n_valid] * right
                        accs[s_idx] = jnp.concatenate(
                            [upd, accs[s_idx][n_valid:]], axis=0)
                else:
                    src = pad1 if dj == 1 else pad2
                    right = src[c, base + 2 + di:base + 2 + di + _TILE_R, :]
                    accs[s_idx] = accs[s_idx] + left * right
        n_acc[base:base + _TILE_R, :] += nacc
        for s_idx in range(len(_SHIFTS)):
            d_acc[s_idx, base:base + _TILE_R, :] += accs[s_idx]

    @pl.when(cc == _NUM_CC - 1)
    def _finalize_image():
        seg = seg_ref[0]
        gtb = gtb_ref[0]
        seg = jnp.where(seg == 255, 0, seg)
        gtb = jnp.where(gtb == 255, 0, gtb)
        r = jax.lax.broadcasted_iota(jnp.int32, (_H, _W), 0)
        c = jax.lax.broadcasted_iota(jnp.int32, (_H, _W), 1)
        interior = (r >= 2) & (r <= _H - 3) & (c >= 2) & (c <= _W - 3)
        maskf = jnp.where((gtb * seg > 0) & interior,
                          jnp.float32(1.0), jnp.float32(0.0))
        inv = jnp.float32(1.0) / jnp.maximum(jnp.sqrt(n_acc[...]),
                                             jnp.float32(_EPS))
        s_total = jnp.float32(0.0)
        for s_idx, (di, dj) in enumerate(_SHIFTS):
            inv_nb = _roll2(inv, di, dj)
            sim = d_acc[s_idx] * inv * inv_nb
            seg_nb = _roll2(seg, di, dj)
            slab = jnp.where(seg == seg_nb, jnp.float32(1.0), jnp.float32(0.0))
            w = maskf + _roll2(maskf, di, dj)
            s_total += jnp.sum(w * (sim - slab) ** 2)
        count = jnp.sum(maskf)
        valid = count >= jnp.float32(1.0)
        contrib = jnp.where(
            valid,
            s_total / (jnp.maximum(count, jnp.float32(1.0)) * jnp.float32(24.0)),
            jnp.float32(0.0))
        s_acc[0] += contrib
        s_acc[1] += jnp.where(valid, jnp.float32(1.0), jnp.float32(0.0))

        @pl.when(b == _B - 1)
        def _finalize_total():
            tot = s_acc[0] / jnp.maximum(s_acc[1], jnp.float32(1.0))
            tot = jnp.where(s_acc[1] == jnp.float32(0.0), jnp.float32(0.0), tot)
            tot = jnp.where(jnp.isnan(tot), jnp.float32(0.0), tot)
            out_ref[...] = jnp.full((1, 1), tot, dtype=jnp.float32)


def kernel(er_input, seg_label, gt_boundary_seg):
    seg_ds = seg_label[:, ::4, ::4]
    gtb_ds = gt_boundary_seg[:, ::4, ::4]

    out = pl.pallas_call(
        _cbl_kernel,
        grid=(_B, _NUM_CC),
        in_specs=[
            pl.BlockSpec((1, _H, _W), lambda b, cc: (b, 0, 0)),
            pl.BlockSpec((1, _H, _W), lambda b, cc: (b, 0, 0)),
            pl.BlockSpec((1, _C_CHUNK, _H, _W), lambda b, cc: (b, cc, 0, 0)),
        ],
        out_specs=pl.BlockSpec((1, 1), lambda b, cc: (0, 0)),
        out_shape=jax.ShapeDtypeStruct((1, 1), jnp.float32),
        scratch_shapes=[
            pltpu.VMEM((len(_SHIFTS), _H, _W), jnp.float32),
            pltpu.VMEM((_H, _W), jnp.float32),
            pltpu.VMEM((_C_CHUNK, _H + 4, _W), jnp.float32),
            pltpu.VMEM((_C_CHUNK, _H + 4, _W), jnp.float32),
            pltpu.VMEM((_C_CHUNK, _H, _W), jnp.float32),
            pltpu.SMEM((2,), jnp.float32),
        ],
    )(seg_ds, gtb_ds, er_input.astype(jnp.bfloat16))
    return out.reshape(())
